# baseline (speedup 1.0000x reference)
import functools
import jax
import jax.numpy as jnp
from jax.experimental import pallas as pl
from jax.experimental.pallas import tpu as pltpu

_POOL_SEGMENTS = 418


def _pad_cols(a, m):
    p = (-a.shape[-1]) % m
    if p:
        a = jnp.pad(a, [(0, 0)] * (a.ndim - 1) + [(0, p)])
    return a


def _deg_kernel(dst_ref, dinv_ref, *, C, NB):
    pid = pl.program_id(0)

    @pl.when(pid == 0)
    def _():
        dinv_ref[:] = jnp.full_like(dinv_ref, 1.0)

    def body(i, carry):
        d = dst_ref[0, 0, i]
        dinv_ref[pl.ds(d, 1), :] = dinv_ref[pl.ds(d, 1), :] + 1.0
        return carry

    jax.lax.fori_loop(0, C, body, 0)

    @pl.when(pid == NB - 1)
    def _():
        dinv_ref[:] = jax.lax.rsqrt(dinv_ref[:])


def _agg_kernel(h_ref, dinv_ref, src_ref, dst_ref, out_ref, *, C, NB):
    pid = pl.program_id(0)

    @pl.when(pid == 0)
    def _():
        out_ref[:] = h_ref[:] * dinv_ref[:, 0:1]

    def body(i, carry):
        s = src_ref[0, 0, i]
        d = dst_ref[0, 0, i]
        g = h_ref[pl.ds(s, 1), :] * dinv_ref[pl.ds(s, 1), 0:1]
        out_ref[pl.ds(d, 1), :] = out_ref[pl.ds(d, 1), :] + g
        return carry

    jax.lax.fori_loop(0, C, body, 0)

    @pl.when(pid == NB - 1)
    def _():
        out_ref[:] = out_ref[:] * dinv_ref[:, 0:1]


def _mm_kernel(x_ref, w_ref, b_ref, o_ref):
    acc = jnp.dot(x_ref[:], w_ref[:], preferred_element_type=jnp.float32)
    o_ref[:] = jnp.maximum(acc + b_ref[:], 0.0)


def _pool_mlp_kernel(h_ref, batch_ref, w1_ref, b1_ref, w2_ref, b2_ref,
                     out_ref, pool_ref, *, C, NB):
    pid = pl.program_id(0)

    @pl.when(pid == 0)
    def _():
        pool_ref[:] = jnp.full_like(pool_ref, -1e30)

    def body(i, carry):
        p = batch_ref[0, 0, i]
        pool_ref[pl.ds(p, 1), :] = jnp.maximum(
            pool_ref[pl.ds(p, 1), :], h_ref[pl.ds(i, 1), :])
        return carry

    jax.lax.fori_loop(0, C, body, 0)

    @pl.when(pid == NB - 1)
    def _():
        pooled = jnp.maximum(pool_ref[:], 0.0)
        h2 = jnp.maximum(
            jnp.dot(pooled, w1_ref[:], preferred_element_type=jnp.float32)
            + b1_ref[:], 0.0)
        logits = jnp.dot(h2, w2_ref[:], preferred_element_type=jnp.float32) + b2_ref[:]
        m = jnp.max(logits, axis=1, keepdims=True)
        lse = m + jnp.log(jnp.sum(jnp.exp(logits - m), axis=1, keepdims=True))
        out_ref[:] = logits - lse


def _deg(dst3, n, C, NB):
    return pl.pallas_call(
        functools.partial(_deg_kernel, C=C, NB=NB),
        grid=(NB,),
        in_specs=[pl.BlockSpec((1, 1, C), lambda e: (e, 0, 0),
                               memory_space=pltpu.SMEM)],
        out_specs=pl.BlockSpec((n, 128), lambda e: (0, 0)),
        out_shape=jax.ShapeDtypeStruct((n, 128), jnp.float32),
    )(dst3)


def _agg(h, dinv, src3, dst3, C, NB):
    n, f = h.shape
    return pl.pallas_call(
        functools.partial(_agg_kernel, C=C, NB=NB),
        grid=(NB,),
        in_specs=[
            pl.BlockSpec((n, f), lambda e: (0, 0)),
            pl.BlockSpec((n, 128), lambda e: (0, 0)),
            pl.BlockSpec((1, 1, C), lambda e: (e, 0, 0), memory_space=pltpu.SMEM),
            pl.BlockSpec((1, 1, C), lambda e: (e, 0, 0), memory_space=pltpu.SMEM),
        ],
        out_specs=pl.BlockSpec((n, f), lambda e: (0, 0)),
        out_shape=jax.ShapeDtypeStruct((n, f), jnp.float32),
    )(h, dinv, src3, dst3)


def _mm_relu(x, w, b):
    n, k = x.shape
    f = w.shape[1]
    r = 1000 if n % 1000 == 0 else n
    nb = n // r
    return pl.pallas_call(
        _mm_kernel,
        grid=(nb,),
        in_specs=[
            pl.BlockSpec((r, k), lambda i: (i, 0)),
            pl.BlockSpec((k, f), lambda i: (0, 0)),
            pl.BlockSpec((1, f), lambda i: (0, 0)),
        ],
        out_specs=pl.BlockSpec((r, f), lambda i: (i, 0)),
        out_shape=jax.ShapeDtypeStruct((n, f), jnp.float32),
    )(x, w, b.reshape(1, f))


def _pool_mlp(h, batch3, w1, b1, w2, b2, pool_pad, C, NB):
    n, f = h.shape
    f2 = w2.shape[1]
    return pl.pallas_call(
        functools.partial(_pool_mlp_kernel, C=C, NB=NB),
        grid=(NB,),
        in_specs=[
            pl.BlockSpec((C, f), lambda i: (i, 0)),
            pl.BlockSpec((1, 1, C), lambda i: (i, 0, 0), memory_space=pltpu.SMEM),
            pl.BlockSpec((f, f), lambda i: (0, 0)),
            pl.BlockSpec((1, f), lambda i: (0, 0)),
            pl.BlockSpec((f, f2), lambda i: (0, 0)),
            pl.BlockSpec((1, f2), lambda i: (0, 0)),
        ],
        out_specs=pl.BlockSpec((pool_pad, f2), lambda i: (0, 0)),
        out_shape=jax.ShapeDtypeStruct((pool_pad, f2), jnp.float32),
        scratch_shapes=[pltpu.VMEM((pool_pad, f), jnp.float32)],
    )(h, batch3, w1, b1.reshape(1, f), w2, b2.reshape(1, f2))


def kernel(x, edge_index, batch, W1, b1, W2, b2, W3, b3,
           lin1_W, lin1_b, lin2_W, lin2_b):
    n = x.shape[0]
    e = edge_index.shape[1]
    hidden = W1.shape[1]
    ncls = lin2_W.shape[1]
    hp = ((hidden + 127) // 128) * 128

    src = edge_index[0]
    dst = edge_index[1]
    ce = 1280 if e % 1280 == 0 else e
    nbe = e // ce
    src3 = src.reshape(nbe, 1, ce)
    dst3 = dst.reshape(nbe, 1, ce)

    dinv = _deg(dst3, n, ce, nbe)

    w1p = _pad_cols(W1, 128)
    b1p = _pad_cols(b1.reshape(1, -1), 128)
    w2p = _pad_cols(jnp.pad(W2, ((0, hp - hidden), (0, 0))), 128)
    b2p = _pad_cols(b2.reshape(1, -1), 128)
    w3p = _pad_cols(jnp.pad(W3, ((0, hp - hidden), (0, 0))), 128)
    b3p = _pad_cols(b3.reshape(1, -1), 128)

    h = _mm_relu(_agg(x, dinv, src3, dst3, ce, nbe), w1p, b1p)
    h = _mm_relu(_agg(h, dinv, src3, dst3, ce, nbe), w2p, b2p)
    h = _mm_relu(_agg(h, dinv, src3, dst3, ce, nbe), w3p, b3p)

    cn = 1000 if n % 1000 == 0 else n
    nbn = n // cn
    batch3 = batch.reshape(nbn, 1, cn)
    pool_pad = ((_POOL_SEGMENTS + 7) // 8) * 8

    l1wp = _pad_cols(jnp.pad(lin1_W, ((0, hp - hidden), (0, 0))), 128)
    l1bp = _pad_cols(lin1_b.reshape(1, -1), 128)
    l2wp = _pad_cols(jnp.pad(lin2_W, ((0, hp - hidden), (0, 0))), 128)
    ncp = l2wp.shape[1]
    l2bp = jnp.concatenate(
        [lin2_b, jnp.full((ncp - ncls,), -1e30, jnp.float32)]).reshape(1, ncp)

    out = _pool_mlp(h, batch3, l1wp, l1bp, l2wp, l2bp, pool_pad, cn, nbn)
    return out[:_POOL_SEGMENTS, :ncls]


# speedup vs baseline: 10.4296x; 10.4296x over previous
import functools
import jax
import jax.numpy as jnp
from jax.experimental import pallas as pl
from jax.experimental.pallas import tpu as pltpu

_POOL_SEGMENTS = 418


def _pad_cols(a, m):
    p = (-a.shape[-1]) % m
    if p:
        a = jnp.pad(a, [(0, 0)] * (a.ndim - 1) + [(0, p)])
    return a


def _deg_kernel(dst_ref, dinv_ref, *, C, NB):
    pid = pl.program_id(0)

    @pl.when(pid == 0)
    def _():
        dinv_ref[:] = jnp.full_like(dinv_ref, 1.0)

    def body(i, carry):
        d = dst_ref[0, 0, i]
        dinv_ref[pl.ds(d, 1), :] = dinv_ref[pl.ds(d, 1), :] + 1.0
        return carry

    jax.lax.fori_loop(0, C, body, 0)

    @pl.when(pid == NB - 1)
    def _():
        dinv_ref[:] = jax.lax.rsqrt(dinv_ref[:])


def _scale_kernel(x_ref, dinv_ref, o_ref):
    o_ref[:] = x_ref[:] * dinv_ref[:, 0:1]


def _agg_kernel(g_ref, src_ref, dst_ref, out_ref, *, C):
    pid = pl.program_id(0)

    @pl.when(pid == 0)
    def _():
        out_ref[:] = g_ref[:]

    def body(i, carry):
        s = src_ref[0, 0, i]
        d = dst_ref[0, 0, i]
        out_ref[pl.ds(d, 1), :] = out_ref[pl.ds(d, 1), :] + g_ref[pl.ds(s, 1), :]
        return carry

    jax.lax.fori_loop(0, C, body, 0, unroll=4)


def _mm_kernel(x_ref, dinv_ref, w_ref, b_ref, o_ref, *, post_scale):
    xs = x_ref[:] * dinv_ref[:, 0:1]
    acc = jnp.dot(xs, w_ref[:], preferred_element_type=jnp.float32)
    acc = jnp.maximum(acc + b_ref[:], 0.0)
    if post_scale:
        acc = acc * dinv_ref[:, 0:1]
    o_ref[:] = acc


def _pool_mlp_kernel(h_ref, batch_ref, w1_ref, b1_ref, w2_ref, b2_ref,
                     out_ref, pool_ref, *, C, NB):
    pid = pl.program_id(0)

    @pl.when(pid == 0)
    def _():
        pool_ref[:] = jnp.full_like(pool_ref, -1e30)

    def body(i, carry):
        p = batch_ref[0, 0, i]
        pool_ref[pl.ds(p, 1), :] = jnp.maximum(
            pool_ref[pl.ds(p, 1), :], h_ref[pl.ds(i, 1), :])
        return carry

    jax.lax.fori_loop(0, C, body, 0)

    @pl.when(pid == NB - 1)
    def _():
        pooled = jnp.maximum(pool_ref[:], 0.0)
        h2 = jnp.maximum(
            jnp.dot(pooled, w1_ref[:], preferred_element_type=jnp.float32)
            + b1_ref[:], 0.0)
        logits = jnp.dot(h2, w2_ref[:], preferred_element_type=jnp.float32) + b2_ref[:]
        m = jnp.max(logits, axis=1, keepdims=True)
        lse = m + jnp.log(jnp.sum(jnp.exp(logits - m), axis=1, keepdims=True))
        out_ref[:] = logits - lse


def _deg(dst3, n, C, NB):
    return pl.pallas_call(
        functools.partial(_deg_kernel, C=C, NB=NB),
        grid=(NB,),
        in_specs=[pl.BlockSpec((1, 1, C), lambda e: (e, 0, 0),
                               memory_space=pltpu.SMEM)],
        out_specs=pl.BlockSpec((n, 128), lambda e: (0, 0)),
        out_shape=jax.ShapeDtypeStruct((n, 128), jnp.float32),
    )(dst3)


def _scale(x, dinv):
    n, f = x.shape
    r = 1000 if n % 1000 == 0 else n
    nb = n // r
    return pl.pallas_call(
        _scale_kernel,
        grid=(nb,),
        in_specs=[
            pl.BlockSpec((r, f), lambda i: (i, 0)),
            pl.BlockSpec((r, 128), lambda i: (i, 0)),
        ],
        out_specs=pl.BlockSpec((r, f), lambda i: (i, 0)),
        out_shape=jax.ShapeDtypeStruct((n, f), jnp.float32),
    )(x, dinv)


def _agg(g, src3, dst3, C, NB):
    n, f = g.shape
    return pl.pallas_call(
        functools.partial(_agg_kernel, C=C),
        grid=(NB,),
        in_specs=[
            pl.BlockSpec((n, f), lambda e: (0, 0)),
            pl.BlockSpec((1, 1, C), lambda e: (e, 0, 0), memory_space=pltpu.SMEM),
            pl.BlockSpec((1, 1, C), lambda e: (e, 0, 0), memory_space=pltpu.SMEM),
        ],
        out_specs=pl.BlockSpec((n, f), lambda e: (0, 0)),
        out_shape=jax.ShapeDtypeStruct((n, f), jnp.float32),
    )(g, src3, dst3)


def _mm_relu(x, dinv, w, b, post_scale):
    n, k = x.shape
    f = w.shape[1]
    r = 1000 if n % 1000 == 0 else n
    nb = n // r
    return pl.pallas_call(
        functools.partial(_mm_kernel, post_scale=post_scale),
        grid=(nb,),
        in_specs=[
            pl.BlockSpec((r, k), lambda i: (i, 0)),
            pl.BlockSpec((r, 128), lambda i: (i, 0)),
            pl.BlockSpec((k, f), lambda i: (0, 0)),
            pl.BlockSpec((1, f), lambda i: (0, 0)),
        ],
        out_specs=pl.BlockSpec((r, f), lambda i: (i, 0)),
        out_shape=jax.ShapeDtypeStruct((n, f), jnp.float32),
    )(x, dinv, w, b.reshape(1, f))


def _pool_mlp(h, batch3, w1, b1, w2, b2, pool_pad, C, NB):
    n, f = h.shape
    f2 = w2.shape[1]
    return pl.pallas_call(
        functools.partial(_pool_mlp_kernel, C=C, NB=NB),
        grid=(NB,),
        in_specs=[
            pl.BlockSpec((C, f), lambda i: (i, 0)),
            pl.BlockSpec((1, 1, C), lambda i: (i, 0, 0), memory_space=pltpu.SMEM),
            pl.BlockSpec((f, f), lambda i: (0, 0)),
            pl.BlockSpec((1, f), lambda i: (0, 0)),
            pl.BlockSpec((f, f2), lambda i: (0, 0)),
            pl.BlockSpec((1, f2), lambda i: (0, 0)),
        ],
        out_specs=pl.BlockSpec((pool_pad, f2), lambda i: (0, 0)),
        out_shape=jax.ShapeDtypeStruct((pool_pad, f2), jnp.float32),
        scratch_shapes=[pltpu.VMEM((pool_pad, f), jnp.float32)],
    )(h, batch3, w1, b1.reshape(1, f), w2, b2.reshape(1, f2))


def kernel(x, edge_index, batch, W1, b1, W2, b2, W3, b3,
           lin1_W, lin1_b, lin2_W, lin2_b):
    n = x.shape[0]
    e = edge_index.shape[1]
    hidden = W1.shape[1]
    ncls = lin2_W.shape[1]
    hp = ((hidden + 127) // 128) * 128

    src = edge_index[0]
    dst = edge_index[1]
    ce = 1280 if e % 1280 == 0 else e
    nbe = e // ce
    src3 = src.reshape(nbe, 1, ce)
    dst3 = dst.reshape(nbe, 1, ce)

    dinv = _deg(dst3, n, ce, nbe)

    w1p = _pad_cols(W1, 128)
    b1p = _pad_cols(b1.reshape(1, -1), 128)
    w2p = _pad_cols(jnp.pad(W2, ((0, hp - hidden), (0, 0))), 128)
    b2p = _pad_cols(b2.reshape(1, -1), 128)
    w3p = _pad_cols(jnp.pad(W3, ((0, hp - hidden), (0, 0))), 128)
    b3p = _pad_cols(b3.reshape(1, -1), 128)

    g = _scale(x, dinv)
    g = _mm_relu(_agg(g, src3, dst3, ce, nbe), dinv, w1p, b1p, True)
    g = _mm_relu(_agg(g, src3, dst3, ce, nbe), dinv, w2p, b2p, True)
    h = _mm_relu(_agg(g, src3, dst3, ce, nbe), dinv, w3p, b3p, False)

    cn = 1000 if n % 1000 == 0 else n
    nbn = n // cn
    batch3 = batch.reshape(nbn, 1, cn)
    pool_pad = ((_POOL_SEGMENTS + 7) // 8) * 8

    l1wp = _pad_cols(jnp.pad(lin1_W, ((0, hp - hidden), (0, 0))), 128)
    l1bp = _pad_cols(lin1_b.reshape(1, -1), 128)
    l2wp = _pad_cols(jnp.pad(lin2_W, ((0, hp - hidden), (0, 0))), 128)
    ncp = l2wp.shape[1]
    l2bp = jnp.concatenate(
        [lin2_b, jnp.full((ncp - ncls,), -1e30, jnp.float32)]).reshape(1, ncp)

    out = _pool_mlp(h, batch3, l1wp, l1bp, l2wp, l2bp, pool_pad, cn, nbn)
    return out[:_POOL_SEGMENTS, :ncls]


# speedup vs baseline: 11.2878x; 1.0823x over previous
import functools
import jax
import jax.numpy as jnp
from jax.experimental import pallas as pl
from jax.experimental.pallas import tpu as pltpu

_POOL_SEGMENTS = 418


def _pad_cols(a, m):
    p = (-a.shape[-1]) % m
    if p:
        a = jnp.pad(a, [(0, 0)] * (a.ndim - 1) + [(0, p)])
    return a


def _deg_kernel(dst_ref, dinv_ref, *, C, NB):
    pid = pl.program_id(0)

    @pl.when(pid == 0)
    def _():
        dinv_ref[:] = jnp.full_like(dinv_ref, 1.0)

    def body(i, carry):
        d = dst_ref[0, 0, i]
        dinv_ref[pl.ds(d, 1), :] = dinv_ref[pl.ds(d, 1), :] + 1.0
        return carry

    jax.lax.fori_loop(0, C, body, 0)

    @pl.when(pid == NB - 1)
    def _():
        dinv_ref[:] = jax.lax.rsqrt(dinv_ref[:])


def _scale_kernel(x_ref, dinv_ref, o_ref):
    o_ref[:] = x_ref[:] * dinv_ref[:, 0:1]


def _agg_kernel(g_ref, src_ref, dst_ref, out_ref, *, C):
    pid = pl.program_id(0)

    @pl.when(pid == 0)
    def _():
        out_ref[:] = g_ref[:]

    def body(i, carry):
        s = src_ref[0, 0, i]
        d = dst_ref[0, 0, i]
        out_ref[pl.ds(d, 1), :] = out_ref[pl.ds(d, 1), :] + g_ref[pl.ds(s, 1), :]
        return carry

    jax.lax.fori_loop(0, C, body, 0, unroll=8)


def _mm_kernel(x_ref, dinv_ref, w_ref, b_ref, o_ref, *, post_scale):
    xs = x_ref[:] * dinv_ref[:, 0:1]
    acc = jnp.dot(xs, w_ref[:], preferred_element_type=jnp.float32)
    acc = jnp.maximum(acc + b_ref[:], 0.0)
    if post_scale:
        acc = acc * dinv_ref[:, 0:1]
    o_ref[:] = acc


def _pool_mlp_kernel(h_ref, batch_ref, w1_ref, b1_ref, w2_ref, b2_ref,
                     out_ref, pool_ref, *, C, NB):
    pid = pl.program_id(0)

    @pl.when(pid == 0)
    def _():
        pool_ref[:] = jnp.full_like(pool_ref, -1e30)

    def body(i, carry):
        p = batch_ref[0, 0, i]
        pool_ref[pl.ds(p, 1), :] = jnp.maximum(
            pool_ref[pl.ds(p, 1), :], h_ref[pl.ds(i, 1), :])
        return carry

    jax.lax.fori_loop(0, C, body, 0)

    @pl.when(pid == NB - 1)
    def _():
        pooled = jnp.maximum(pool_ref[:], 0.0)
        h2 = jnp.maximum(
            jnp.dot(pooled, w1_ref[:], preferred_element_type=jnp.float32)
            + b1_ref[:], 0.0)
        logits = jnp.dot(h2, w2_ref[:], preferred_element_type=jnp.float32) + b2_ref[:]
        m = jnp.max(logits, axis=1, keepdims=True)
        lse = m + jnp.log(jnp.sum(jnp.exp(logits - m), axis=1, keepdims=True))
        out_ref[:] = logits - lse


def _deg(dst3, n, C, NB):
    return pl.pallas_call(
        functools.partial(_deg_kernel, C=C, NB=NB),
        grid=(NB,),
        in_specs=[pl.BlockSpec((1, 1, C), lambda e: (e, 0, 0),
                               memory_space=pltpu.SMEM)],
        out_specs=pl.BlockSpec((n, 128), lambda e: (0, 0)),
        out_shape=jax.ShapeDtypeStruct((n, 128), jnp.float32),
    )(dst3)


def _scale(x, dinv):
    n, f = x.shape
    r = 1000 if n % 1000 == 0 else n
    nb = n // r
    return pl.pallas_call(
        _scale_kernel,
        grid=(nb,),
        in_specs=[
            pl.BlockSpec((r, f), lambda i: (i, 0)),
            pl.BlockSpec((r, 128), lambda i: (i, 0)),
        ],
        out_specs=pl.BlockSpec((r, f), lambda i: (i, 0)),
        out_shape=jax.ShapeDtypeStruct((n, f), jnp.float32),
    )(x, dinv)


def _agg(g, src3, dst3, C, NB):
    n, f = g.shape
    return pl.pallas_call(
        functools.partial(_agg_kernel, C=C),
        grid=(NB,),
        in_specs=[
            pl.BlockSpec((n, f), lambda e: (0, 0)),
            pl.BlockSpec((1, 1, C), lambda e: (e, 0, 0), memory_space=pltpu.SMEM),
            pl.BlockSpec((1, 1, C), lambda e: (e, 0, 0), memory_space=pltpu.SMEM),
        ],
        out_specs=pl.BlockSpec((n, f), lambda e: (0, 0)),
        out_shape=jax.ShapeDtypeStruct((n, f), jnp.float32),
    )(g, src3, dst3)


def _mm_relu(x, dinv, w, b, post_scale):
    n, k = x.shape
    f = w.shape[1]
    r = 1000 if n % 1000 == 0 else n
    nb = n // r
    return pl.pallas_call(
        functools.partial(_mm_kernel, post_scale=post_scale),
        grid=(nb,),
        in_specs=[
            pl.BlockSpec((r, k), lambda i: (i, 0)),
            pl.BlockSpec((r, 128), lambda i: (i, 0)),
            pl.BlockSpec((k, f), lambda i: (0, 0)),
            pl.BlockSpec((1, f), lambda i: (0, 0)),
        ],
        out_specs=pl.BlockSpec((r, f), lambda i: (i, 0)),
        out_shape=jax.ShapeDtypeStruct((n, f), jnp.float32),
    )(x, dinv, w, b.reshape(1, f))


def _pool_mlp(h, batch3, w1, b1, w2, b2, pool_pad, C, NB):
    n, f = h.shape
    f2 = w2.shape[1]
    return pl.pallas_call(
        functools.partial(_pool_mlp_kernel, C=C, NB=NB),
        grid=(NB,),
        in_specs=[
            pl.BlockSpec((C, f), lambda i: (i, 0)),
            pl.BlockSpec((1, 1, C), lambda i: (i, 0, 0), memory_space=pltpu.SMEM),
            pl.BlockSpec((f, f), lambda i: (0, 0)),
            pl.BlockSpec((1, f), lambda i: (0, 0)),
            pl.BlockSpec((f, f2), lambda i: (0, 0)),
            pl.BlockSpec((1, f2), lambda i: (0, 0)),
        ],
        out_specs=pl.BlockSpec((pool_pad, f2), lambda i: (0, 0)),
        out_shape=jax.ShapeDtypeStruct((pool_pad, f2), jnp.float32),
        scratch_shapes=[pltpu.VMEM((pool_pad, f), jnp.float32)],
    )(h, batch3, w1, b1.reshape(1, f), w2, b2.reshape(1, f2))


def kernel(x, edge_index, batch, W1, b1, W2, b2, W3, b3,
           lin1_W, lin1_b, lin2_W, lin2_b):
    n = x.shape[0]
    e = edge_index.shape[1]
    hidden = W1.shape[1]
    ncls = lin2_W.shape[1]
    hp = ((hidden + 127) // 128) * 128

    src = edge_index[0]
    dst = edge_index[1]
    ce = 1280 if e % 1280 == 0 else e
    nbe = e // ce
    src3 = src.reshape(nbe, 1, ce)
    dst3 = dst.reshape(nbe, 1, ce)

    dinv = _deg(dst3, n, ce, nbe)

    w1p = _pad_cols(W1, 128)
    b1p = _pad_cols(b1.reshape(1, -1), 128)
    w2p = _pad_cols(jnp.pad(W2, ((0, hp - hidden), (0, 0))), 128)
    b2p = _pad_cols(b2.reshape(1, -1), 128)
    w3p = _pad_cols(jnp.pad(W3, ((0, hp - hidden), (0, 0))), 128)
    b3p = _pad_cols(b3.reshape(1, -1), 128)

    g = _scale(x, dinv)
    g = _mm_relu(_agg(g, src3, dst3, ce, nbe), dinv, w1p, b1p, True)
    g = _mm_relu(_agg(g, src3, dst3, ce, nbe), dinv, w2p, b2p, True)
    h = _mm_relu(_agg(g, src3, dst3, ce, nbe), dinv, w3p, b3p, False)

    cn = 1000 if n % 1000 == 0 else n
    nbn = n // cn
    batch3 = batch.reshape(nbn, 1, cn)
    pool_pad = ((_POOL_SEGMENTS + 7) // 8) * 8

    l1wp = _pad_cols(jnp.pad(lin1_W, ((0, hp - hidden), (0, 0))), 128)
    l1bp = _pad_cols(lin1_b.reshape(1, -1), 128)
    l2wp = _pad_cols(jnp.pad(lin2_W, ((0, hp - hidden), (0, 0))), 128)
    ncp = l2wp.shape[1]
    l2bp = jnp.concatenate(
        [lin2_b, jnp.full((ncp - ncls,), -1e30, jnp.float32)]).reshape(1, ncp)

    out = _pool_mlp(h, batch3, l1wp, l1bp, l2wp, l2bp, pool_pad, cn, nbn)
    return out[:_POOL_SEGMENTS, :ncls]
